# Initial kernel scaffold; baseline (speedup 1.0000x reference)
import jax
import jax.numpy as jnp
from jax.experimental import pallas as pl


def kernel(x, kge_emb, W1_0, b1_0, W2_0, b2_0, W1_1, b1_1, W2_1, b2_1, Wo, bo):
    raise NotImplementedError("write your pallas kernel here")


# speedup vs baseline: 14.4835x; 14.4835x over previous
import functools

import jax
import jax.numpy as jnp
from jax import lax
from jax.experimental import pallas as pl
from jax.experimental.pallas import tpu as pltpu
from jax.experimental.pallas import tpu_sc as plsc

_TOPK = 64
_BLK = 128
_CAP = 256
_NW = 32


def _mlp(x, W1s, b1s, W2s, b2s, rb):
    b, din = x.shape
    d = W2s.shape[-1]

    def body(x_ref, w1_ref, b1_ref, w2_ref, b2_ref, o_ref):
        h = jnp.dot(x_ref[...], w1_ref[0], preferred_element_type=jnp.float32)
        h = jnp.maximum(h + b1_ref[0], 0.0)
        sp = jnp.dot(h, w2_ref[0], preferred_element_type=jnp.float32)
        o_ref[...] = sp + b2_ref[0]

    return pl.pallas_call(
        body,
        grid=(2, b // rb),
        in_specs=[
            pl.BlockSpec((rb, din), lambda h, i: (i, 0)),
            pl.BlockSpec((1, din, din), lambda h, i: (h, 0, 0)),
            pl.BlockSpec((1, 1, din), lambda h, i: (h, 0, 0)),
            pl.BlockSpec((1, din, d), lambda h, i: (h, 0, 0)),
            pl.BlockSpec((1, 1, d), lambda h, i: (h, 0, 0)),
        ],
        out_specs=pl.BlockSpec((rb, d), lambda h, i: (h * (b // rb) + i, 0)),
        out_shape=jax.ShapeDtypeStruct((2 * b, d), jnp.float32),
    )(x, W1s, b1s, W2s, b2s)


def _score(spanned, kge, rb, cb):
    hb, d = spanned.shape
    k = kge.shape[0]
    nblk = k // _BLK

    def body(sp_ref, kge_ref, s_ref, m_ref):
        s = lax.dot_general(sp_ref[...], kge_ref[...], (((1,), (1,)), ((), ())),
                            preferred_element_type=jnp.float32)
        s3 = s.reshape(rb, cb // _BLK, _BLK)
        s_ref[...] = s3
        m_ref[...] = jnp.max(s3, axis=-1)[None]

    return pl.pallas_call(
        body,
        grid=(k // cb, hb // rb),
        in_specs=[
            pl.BlockSpec((rb, d), lambda j, i: (i, 0)),
            pl.BlockSpec((cb, d), lambda j, i: (j, 0)),
        ],
        out_specs=[
            pl.BlockSpec((rb, cb // _BLK, _BLK), lambda j, i: (i, j, 0)),
            pl.BlockSpec((1, rb, cb // _BLK), lambda j, i: (j, i, 0)),
        ],
        out_shape=[
            jax.ShapeDtypeStruct((hb, nblk, _BLK), jnp.float32),
            jax.ShapeDtypeStruct((k // cb, hb, cb // _BLK), jnp.float32),
        ],
    )(spanned, kge)


def _blocksel(bmax, rb):
    hb, nblk = bmax.shape

    def body(m_ref, aux_ref, thr_ref):
        arr = m_ref[...]
        lanes = lax.broadcasted_iota(jnp.int32, (rb, nblk), 1)
        k64 = lax.broadcasted_iota(jnp.int32, (rb, _TOPK), 1)

        def round_(r, carry):
            arr, bidx, _ = carry
            m = jnp.max(arr, axis=1)
            pos = jnp.min(jnp.where(arr == m[:, None], lanes, nblk), axis=1)
            onehot = lanes == pos[:, None]
            arr = jnp.where(onehot, -jnp.inf, arr)
            bidx = jnp.where(k64 == r, pos[:, None], bidx)
            return arr, bidx, m

        init = (arr, jnp.zeros((rb, _TOPK), jnp.int32), jnp.zeros((rb,), jnp.float32))
        _, bidx, t = lax.fori_loop(0, _TOPK, round_, init)
        aux_ref[...] = bidx
        thr_ref[...] = jnp.broadcast_to(t[:, None], (rb, 16))

    return pl.pallas_call(
        body,
        grid=(hb // rb,),
        in_specs=[pl.BlockSpec((rb, nblk), lambda i: (i, 0))],
        out_specs=[
            pl.BlockSpec((rb, _TOPK), lambda i: (i, 0)),
            pl.BlockSpec((rb, 16), lambda i: (i, 0)),
        ],
        out_shape=[
            jax.ShapeDtypeStruct((hb, _TOPK), jnp.int32),
            jax.ShapeDtypeStruct((hb, 16), jnp.float32),
        ],
    )(bmax)


def _sc_compact(scores_flat, bidx, thr, nblk):
    hb = bidx.shape[0]
    tpw = hb // _NW
    mesh = plsc.VectorSubcoreMesh(core_axis_name="c", subcore_axis_name="s",
                                  num_cores=2, num_subcores=16)

    @functools.partial(
        pl.kernel,
        out_type=[
            jax.ShapeDtypeStruct((hb, _CAP), jnp.float32),
            jax.ShapeDtypeStruct((hb, _CAP), jnp.int32),
        ],
        mesh=mesh,
        compiler_params=pltpu.CompilerParams(needs_layout_passes=False),
        scratch_types=[
            pltpu.VMEM((_TOPK,), jnp.int32),
            pltpu.VMEM((16,), jnp.float32),
            pltpu.VMEM((_TOPK,), jnp.int32),
            pltpu.VMEM((_TOPK, _BLK), jnp.float32),
            pltpu.VMEM((_CAP,), jnp.float32),
            pltpu.VMEM((_CAP,), jnp.int32),
            pltpu.SemaphoreType.DMA,
        ],
    )
    def k(scores_hbm, bidx_hbm, thr_hbm, cval_hbm, cidx_hbm, bidxv, thrv,
          gbuf, candsc, cval, cidx, sem):
        wid = lax.axis_index("s") * 2 + lax.axis_index("c")
        iota = lax.iota(jnp.int32, 16)

        def task(tt, _):
            r = wid * tpw + tt
            pltpu.sync_copy(bidx_hbm.at[r], bidxv)
            pltpu.sync_copy(thr_hbm.at[r], thrv)
            for i in range(4):
                gbuf[pl.ds(i * 16, 16)] = bidxv[pl.ds(i * 16, 16)] + r * nblk
            pltpu.async_copy(scores_hbm.at[gbuf], candsc, sem).wait()
            tvec = thrv[...]
            for i in range(_CAP // 16):
                cval[pl.ds(i * 16, 16)] = jnp.full((16,), -1e30, jnp.float32)
                cidx[pl.ds(i * 16, 16)] = jnp.zeros((16,), jnp.int32)

            def blkbody(j, base):
                blk = j // 8
                sub = j - blk * 8
                v = candsc[blk, pl.ds(sub * 16, 16)]
                mask = v >= tvec
                cs = plsc.cumsum(jnp.where(mask, 1, 0).astype(jnp.int32))
                pos = jnp.minimum(base + cs - 1, _CAP - 1)
                gidx = j * 16 + iota
                plsc.store_scatter(cval, [pos], v, mask=mask)
                plsc.store_scatter(cidx, [pos], gidx, mask=mask)
                return base + plsc.all_reduce_population_count(mask)

            lax.fori_loop(0, _TOPK * 8, blkbody, jnp.zeros((16,), jnp.int32))
            pltpu.sync_copy(cval, cval_hbm.at[r])
            pltpu.sync_copy(cidx, cidx_hbm.at[r])
            return 0

        lax.fori_loop(0, tpw, task, 0)

    return k(scores_flat, bidx, thr)


def _select(cval2, cidx2, bidx, rb):
    hb = cval2.shape[0]

    def body(cv_ref, ci_ref, bx_ref, ti_ref, w_ref):
        v = cv_ref[...]
        ci = ci_ref[...]
        lanes = lax.broadcasted_iota(jnp.int32, (rb, _CAP), 1)
        k64 = lax.broadcasted_iota(jnp.int32, (rb, _TOPK), 1)

        def round_(r, carry):
            v, tv, ti = carry
            m = jnp.max(v, axis=1)
            pos = jnp.min(jnp.where(v == m[:, None], lanes, _CAP), axis=1)
            onehot = lanes == pos[:, None]
            idx_r = jnp.sum(jnp.where(onehot, ci, 0), axis=1)
            v = jnp.where(onehot, -jnp.inf, v)
            tv = jnp.where(k64 == r, m[:, None], tv)
            ti = jnp.where(k64 == r, idx_r[:, None], ti)
            return v, tv, ti

        init = (v, jnp.zeros((rb, _TOPK), jnp.float32),
                jnp.zeros((rb, _TOPK), jnp.int32))
        _, tv, ti = lax.fori_loop(0, _TOPK, round_, init)
        e = jnp.exp(tv)
        u = jnp.exp(e - jnp.max(e, axis=1, keepdims=True))
        w = u / jnp.sum(u, axis=1, keepdims=True)
        bx = bx_ref[...]
        blk_of = ti >> 7
        low = ti & 127
        gb = jnp.zeros((rb, _TOPK), jnp.int32)
        for j in range(_TOPK):
            gb = gb + jnp.where(blk_of == j, bx[:, j][:, None], 0)
        ti_ref[...] = gb * _BLK + low
        w_ref[...] = w

    return pl.pallas_call(
        body,
        grid=(hb // rb,),
        in_specs=[
            pl.BlockSpec((rb, _CAP), lambda i: (i, 0)),
            pl.BlockSpec((rb, _CAP), lambda i: (i, 0)),
            pl.BlockSpec((rb, _TOPK), lambda i: (i, 0)),
        ],
        out_specs=[
            pl.BlockSpec((rb, _TOPK), lambda i: (i, 0)),
            pl.BlockSpec((rb, _TOPK), lambda i: (i, 0)),
        ],
        out_shape=[
            jax.ShapeDtypeStruct((hb, _TOPK), jnp.int32),
            jax.ShapeDtypeStruct((hb, _TOPK), jnp.float32),
        ],
    )(cval2, cidx2, bidx)


def _sc_gather(ti, kge):
    hb = ti.shape[0]
    d = kge.shape[1]
    tpw = hb // _NW
    mesh = plsc.VectorSubcoreMesh(core_axis_name="c", subcore_axis_name="s",
                                  num_cores=2, num_subcores=16)

    @functools.partial(
        pl.kernel,
        out_type=jax.ShapeDtypeStruct((hb, _TOPK, d), jnp.float32),
        mesh=mesh,
        compiler_params=pltpu.CompilerParams(needs_layout_passes=False),
        scratch_types=[
            pltpu.VMEM((_TOPK,), jnp.int32),
            pltpu.VMEM((_TOPK, d), jnp.float32),
            pltpu.SemaphoreType.DMA,
        ],
    )
    def k(ti_hbm, kge_hbm, gath_hbm, idxv, krows, sem):
        wid = lax.axis_index("s") * 2 + lax.axis_index("c")

        def task(tt, _):
            r = wid * tpw + tt
            pltpu.sync_copy(ti_hbm.at[r], idxv)
            pltpu.async_copy(kge_hbm.at[idxv], krows, sem).wait()
            pltpu.sync_copy(krows, gath_hbm.at[r])
            return 0

        lax.fori_loop(0, tpw, task, 0)

    return k(ti, kge)


def _wsum(gath, w, rb):
    hb, topk, d = gath.shape

    def body(g_ref, w_ref, o_ref):
        g = g_ref[...]
        wt = w_ref[...]
        o_ref[...] = jnp.sum(g * wt[:, :, None], axis=1)

    return pl.pallas_call(
        body,
        grid=(hb // rb,),
        in_specs=[
            pl.BlockSpec((rb, topk, d), lambda i: (i, 0, 0)),
            pl.BlockSpec((rb, topk), lambda i: (i, 0)),
        ],
        out_specs=pl.BlockSpec((rb, d), lambda i: (i, 0)),
        out_shape=jax.ShapeDtypeStruct((hb, d), jnp.float32),
    )(gath, w)


def _proj(vals, Wo, bo2):
    hb, d = vals.shape
    b = hb // 2

    def body(v_ref, wo_ref, bo_ref, o_ref):
        v = v_ref[...]
        o = jnp.dot(v[:b], wo_ref[:d], preferred_element_type=jnp.float32)
        o = o + jnp.dot(v[b:], wo_ref[d:], preferred_element_type=jnp.float32)
        o_ref[...] = o + bo_ref[0][None, :]

    return pl.pallas_call(
        body,
        grid=(1,),
        in_specs=[
            pl.BlockSpec((hb, d), lambda i: (0, 0)),
            pl.BlockSpec((2 * d, d), lambda i: (0, 0)),
            pl.BlockSpec((1, d), lambda i: (0, 0)),
        ],
        out_specs=pl.BlockSpec((b, d), lambda i: (0, 0)),
        out_shape=jax.ShapeDtypeStruct((b, d), jnp.float32),
    )(vals, Wo, bo2)


def kernel(x, kge_emb, W1_0, b1_0, W2_0, b2_0, W1_1, b1_1, W2_1, b2_1, Wo, bo):
    b = x.shape[0]
    k, d = kge_emb.shape
    nblk = k // _BLK
    rb = min(256, b)

    W1s = jnp.stack([W1_0, W1_1])
    b1s = jnp.stack([b1_0, b1_1])[:, None, :]
    W2s = jnp.stack([W2_0, W2_1])
    b2s = jnp.stack([b2_0, b2_1])[:, None, :]

    spanned = _mlp(x, W1s, b1s, W2s, b2s, rb)
    scores3, bmax3 = _score(spanned, kge_emb, rb, 2048)
    bmax = bmax3.transpose(1, 0, 2).reshape(2 * b, nblk)
    bidx, thr = _blocksel(bmax, rb)
    scores_flat = scores3.reshape(2 * b * nblk, _BLK)
    cval2, cidx2 = _sc_compact(scores_flat, bidx, thr, nblk)
    ti, w = _select(cval2, cidx2, bidx, rb)
    gath = _sc_gather(ti, kge_emb)
    vals = _wsum(gath, w, 64)
    return _proj(vals, Wo, bo.reshape(1, d))
